# baseline (speedup 1.0000x reference)
import jax
import jax.numpy as jnp
from jax import lax
from jax.experimental import pallas as pl
from jax.experimental.pallas import tpu as pltpu
from jax.experimental.pallas import tpu_sc as plsc

N = 10000
D = 128
E = 320000
NC, NS = 2, 16
NW = NC * NS
CH = 64
NCH0 = 159
NCH1 = 159
NCHM = max(NCH0, NCH1)
E0 = NS * NCH0 * CH
NPAD = 10112
SLAB = NPAD // NS


def _mm(x, w):
    def body(x_ref, w_ref, o_ref):
        o_ref[...] = jnp.dot(x_ref[...], w_ref[...],
                             preferred_element_type=jnp.float32)
    return pl.pallas_call(
        body,
        grid=(10,),
        in_specs=[pl.BlockSpec((N // 10, D), lambda i: (i, 0)),
                  pl.BlockSpec((D, D), lambda i: (0, 0))],
        out_specs=pl.BlockSpec((N // 10, D), lambda i: (i, 0)),
        out_shape=jax.ShapeDtypeStruct((N, D), jnp.float32),
    )(x, w)


def _cmm(a, b, w):
    def body(a_ref, b_ref, w_ref, o_ref):
        h = jnp.maximum(a_ref[0] + a_ref[1] + b_ref[...], 0.0)
        o_ref[...] = jnp.dot(h, w_ref[...], preferred_element_type=jnp.float32)
    return pl.pallas_call(
        body,
        grid=(10,),
        in_specs=[pl.BlockSpec((2, N // 10, D), lambda i: (0, i, 0)),
                  pl.BlockSpec((1, D), lambda i: (0, 0)),
                  pl.BlockSpec((D, D), lambda i: (0, 0))],
        out_specs=pl.BlockSpec((N // 10, D), lambda i: (i, 0)),
        out_shape=jax.ShapeDtypeStruct((N, D), jnp.float32),
    )(a, b, w)


def _final(a, b):
    def body(a_ref, b_ref, o_ref):
        o_ref[...] = jnp.maximum(a_ref[0] + a_ref[1] + b_ref[...], 0.0)
    return pl.pallas_call(
        body,
        grid=(10,),
        in_specs=[pl.BlockSpec((2, N // 10, D), lambda i: (0, i, 0)),
                  pl.BlockSpec((1, D), lambda i: (0, 0))],
        out_specs=pl.BlockSpec((N // 10, D), lambda i: (i, 0)),
        out_shape=jax.ShapeDtypeStruct((N, D), jnp.float32),
    )(a, b)


def _sc_body(h_hbm, meta_hbm, dst_hbm, out_hbm,
             meta_v, dst_v, rows_v0, rows_v1, rows_v2, acc_s,
             gs0, gs1, gs2, ss0, ss1, ss2, ms0, ms1, ms2, ds0, ds1, ds2):
    c = lax.axis_index("c")
    s = lax.axis_index("s")
    w = c * NS + s
    nch = jnp.where(c == 0, NCH0, NCH1)
    rows = (rows_v0, rows_v1, rows_v2)
    gs = (gs0, gs1, gs2)
    ss = (ss0, ss1, ss2)
    ms = (ms0, ms1, ms2)
    ds = (ds0, ds1, ds2)

    def zb(i, carry):
        for k in range(D // 16):
            rows_v0[i, pl.ds(k * 16, 16)] = jnp.zeros((16,), jnp.float32)
        return carry
    lax.fori_loop(0, CH, zb, 0)
    for r in range(SLAB // CH):
        pltpu.sync_copy(rows_v0, acc_s.at[pl.ds(s * SLAB + r * CH, CH)])
    rem = SLAB % CH
    if rem:
        pltpu.sync_copy(rows_v0.at[pl.ds(0, rem)],
                        acc_s.at[pl.ds(s * SLAB + (SLAB // CH) * CH, rem)])
    plsc.subcore_barrier()

    def scale(rows_v, b):
        @plsc.parallel_loop(0, CH, step=1, unroll=4)
        def eb(e):
            ewb = (meta_v[b, 1 + (e >> 2), pl.ds((e & 3) * 16, 16)]
                   .astype(jnp.float32) * jnp.float32(2.0 ** -24))
            for k in range(D // 16):
                sl = pl.ds(k * 16, 16)
                rows_v[e, sl] = rows_v[e, sl] * ewb

    pltpu.sync_copy(meta_hbm.at[w, 0], meta_v.at[0])
    pltpu.sync_copy(meta_hbm.at[w, 1], meta_v.at[1])
    pltpu.sync_copy(dst_hbm.at[w, 0], dst_v.at[0])
    pltpu.sync_copy(dst_hbm.at[w, 1], dst_v.at[1])
    pltpu.async_copy(h_hbm.at[meta_v.at[0, 0]], rows_v0, gs0)
    pltpu.async_copy(h_hbm.at[meta_v.at[1, 0]], rows_v1, gs1)
    pltpu.async_copy(meta_hbm.at[w, 2], meta_v.at[2], ms2)

    def step(jj, carry):
        for b in range(3):
            k = 3 * jj + b
            bp = (b + 2) % 3

            @pl.when(k > 0)
            def _():
                pltpu.make_async_copy(rows[bp], acc_s.at[dst_v.at[bp]],
                                      ss[bp]).wait()

            @pl.when(k + 2 < nch)
            def _():
                pltpu.async_copy(dst_hbm.at[w, k + 2], dst_v.at[bp], ds[bp])
                pltpu.make_async_copy(meta_hbm.at[w, k + 2], meta_v.at[bp],
                                      ms[bp]).wait()
                pltpu.async_copy(h_hbm.at[meta_v.at[bp, 0]], rows[bp], gs[bp])
            pltpu.make_async_copy(h_hbm.at[meta_v.at[b, 0]], rows[b],
                                  gs[b]).wait()
            scale(rows[b], b)

            @pl.when(k + 3 < nch)
            def _():
                pltpu.async_copy(meta_hbm.at[w, k + 3], meta_v.at[b], ms[b])

            @pl.when(k > 1)
            def _():
                pltpu.make_async_copy(dst_hbm.at[w, k], dst_v.at[b],
                                      ds[b]).wait()
            pltpu.async_copy(rows[b], acc_s.at[dst_v.at[b]], ss[b], add=True)
        return carry
    lax.fori_loop(0, nch // 3, step, 0)

    pltpu.make_async_copy(rows_v2, acc_s.at[dst_v.at[2]], ss2).wait()

    plsc.subcore_barrier()
    pltpu.sync_copy(acc_s.at[pl.ds(s * SLAB, SLAB)],
                    out_hbm.at[c, pl.ds(s * SLAB, SLAB)])


MR = 1 + CH // 4


def _sc_scatter(h, metap, dstp):
    mesh = plsc.VectorSubcoreMesh(core_axis_name="c", subcore_axis_name="s",
                                  num_cores=NC, num_subcores=NS)
    k = pl.kernel(
        _sc_body,
        out_type=jax.ShapeDtypeStruct((NC, NPAD, D), jnp.float32),
        mesh=mesh,
        scratch_types=[
            pltpu.VMEM((3, MR, CH), jnp.int32),
            pltpu.VMEM((3, CH), jnp.int32),
            pltpu.VMEM((CH, D), jnp.float32),
            pltpu.VMEM((CH, D), jnp.float32),
            pltpu.VMEM((CH, D), jnp.float32),
            pltpu.VMEM_SHARED((NPAD, D), jnp.float32),
            pltpu.SemaphoreType.DMA,
            pltpu.SemaphoreType.DMA,
            pltpu.SemaphoreType.DMA,
            pltpu.SemaphoreType.DMA,
            pltpu.SemaphoreType.DMA,
            pltpu.SemaphoreType.DMA,
            pltpu.SemaphoreType.DMA,
            pltpu.SemaphoreType.DMA,
            pltpu.SemaphoreType.DMA,
            pltpu.SemaphoreType.DMA,
            pltpu.SemaphoreType.DMA,
            pltpu.SemaphoreType.DMA,
        ],
    )
    return k(h, metap, dstp)


def _split_edges(arr):
    a0 = arr[:E0].reshape(NS, NCH0, CH)
    a0 = jnp.pad(a0, ((0, 0), (0, NCHM - NCH0), (0, 0)))
    per1 = (E - E0) // NS
    a1 = arr[E0:].reshape(NS, per1)
    a1 = jnp.pad(a1, ((0, 0), (0, NCH1 * CH - per1))).reshape(NS, NCH1, CH)
    a1 = jnp.pad(a1, ((0, 0), (0, NCHM - NCH1), (0, 0)))
    return jnp.concatenate([a0, a1], axis=0)


def kernel(x, edge_idx, edge_attr, W0, b0, W1, b1, W2, b2):
    srcp = _split_edges(edge_idx[0])
    dstp = _split_edges(edge_idx[1])
    ewfx = jnp.round(_split_edges(edge_attr) * jnp.float32(2.0 ** 24)
                     ).astype(jnp.int32)
    ewrep = jnp.broadcast_to(ewfx[..., None], (NW, NCHM, CH, 16)).reshape(
        NW, NCHM, MR - 1, CH) + jnp.zeros((), jnp.int32)
    metap = jnp.concatenate([srcp[:, :, None, :], ewrep], axis=2)
    b0r = b0.reshape(1, D)
    b1r = b1.reshape(1, D)
    b2r = b2.reshape(1, D)

    h = _mm(x, W0)
    a = _sc_scatter(h, metap, dstp)
    h = _cmm(a, b0r, W1)
    a = _sc_scatter(h, metap, dstp)
    h = _cmm(a, b1r, W2)
    a = _sc_scatter(h, metap, dstp)
    return _final(a, b2r)


# speedup vs baseline: 1.0579x; 1.0579x over previous
import jax
import jax.numpy as jnp
from jax import lax
from jax.experimental import pallas as pl
from jax.experimental.pallas import tpu as pltpu
from jax.experimental.pallas import tpu_sc as plsc

N = 10000
D = 128
E = 320000
NC, NS = 2, 16
NW = NC * NS
CH = 64
NCH0 = 158
NCH1 = 158
NCHM = max(NCH0, NCH1)
E0 = NS * NCH0 * CH
NPAD = 10112
SLAB = NPAD // NS


def _mm(x, w):
    def body(x_ref, w_ref, o_ref):
        o_ref[...] = jnp.dot(x_ref[...], w_ref[...],
                             preferred_element_type=jnp.float32)
    return pl.pallas_call(
        body,
        grid=(10,),
        in_specs=[pl.BlockSpec((N // 10, D), lambda i: (i, 0)),
                  pl.BlockSpec((D, D), lambda i: (0, 0))],
        out_specs=pl.BlockSpec((N // 10, D), lambda i: (i, 0)),
        out_shape=jax.ShapeDtypeStruct((N, D), jnp.float32),
    )(x, w)


def _cmm(a, b, w):
    def body(a_ref, b_ref, w_ref, o_ref):
        h = jnp.maximum(a_ref[0] + a_ref[1] + b_ref[...], 0.0)
        o_ref[...] = jnp.dot(h, w_ref[...], preferred_element_type=jnp.float32)
    return pl.pallas_call(
        body,
        grid=(10,),
        in_specs=[pl.BlockSpec((2, N // 10, D), lambda i: (0, i, 0)),
                  pl.BlockSpec((1, D), lambda i: (0, 0)),
                  pl.BlockSpec((D, D), lambda i: (0, 0))],
        out_specs=pl.BlockSpec((N // 10, D), lambda i: (i, 0)),
        out_shape=jax.ShapeDtypeStruct((N, D), jnp.float32),
    )(a, b, w)


def _final(a, b):
    def body(a_ref, b_ref, o_ref):
        o_ref[...] = jnp.maximum(a_ref[0] + a_ref[1] + b_ref[...], 0.0)
    return pl.pallas_call(
        body,
        grid=(10,),
        in_specs=[pl.BlockSpec((2, N // 10, D), lambda i: (0, i, 0)),
                  pl.BlockSpec((1, D), lambda i: (0, 0))],
        out_specs=pl.BlockSpec((N // 10, D), lambda i: (i, 0)),
        out_shape=jax.ShapeDtypeStruct((N, D), jnp.float32),
    )(a, b)


def _sc_body(h_hbm, src_hbm, dst_hbm, ew_hbm, out_hbm,
             src_v0, src_v1, dst_v, ew_v0, ew_v1, rows_v0, rows_v1, acc_s,
             gs0, gs1, ss0, ss1, is0, is1, es0, es1, ds0, ds1):
    c = lax.axis_index("c")
    s = lax.axis_index("s")
    w = c * NS + s
    nch = jnp.where(c == 0, NCH0, NCH1)

    def zb(i, carry):
        for k in range(D // 16):
            rows_v0[i, pl.ds(k * 16, 16)] = jnp.zeros((16,), jnp.float32)
        return carry
    lax.fori_loop(0, CH, zb, 0)
    for r in range(SLAB // CH):
        pltpu.sync_copy(rows_v0, acc_s.at[pl.ds(s * SLAB + r * CH, CH)])
    rem = SLAB % CH
    if rem:
        pltpu.sync_copy(rows_v0.at[pl.ds(0, rem)],
                        acc_s.at[pl.ds(s * SLAB + (SLAB // CH) * CH, rem)])
    plsc.subcore_barrier()

    pltpu.sync_copy(dst_hbm.at[w, 0], dst_v.at[0])

    def scale(rows_v, ew_v):
        @plsc.parallel_loop(0, CH, step=1, unroll=4)
        def eb(e):
            ewb = ew_v[e, :]
            for k in range(D // 16):
                sl = pl.ds(k * 16, 16)
                rows_v[e, sl] = rows_v[e, sl] * ewb

    pltpu.sync_copy(src_hbm.at[w, 0], src_v0)
    pltpu.sync_copy(ew_hbm.at[w, 0], ew_v0)
    pltpu.async_copy(h_hbm.at[src_v0], rows_v0, gs0)
    pltpu.async_copy(src_hbm.at[w, 1], src_v1, is1)
    pltpu.async_copy(ew_hbm.at[w, 1], ew_v1, es1)

    def step(jj, carry):
        j0 = 2 * jj
        j1 = j0 + 1
        pltpu.make_async_copy(src_hbm.at[w, j1], src_v1, is1).wait()
        pltpu.make_async_copy(ew_hbm.at[w, j1], ew_v1, es1).wait()

        @pl.when(jj > 0)
        def _():
            pltpu.make_async_copy(rows_v1, acc_s.at[dst_v.at[1]], ss1).wait()
        pltpu.async_copy(dst_hbm.at[w, j1], dst_v.at[1], ds1)
        pltpu.async_copy(h_hbm.at[src_v1], rows_v1, gs1)
        pltpu.make_async_copy(h_hbm.at[src_v0], rows_v0, gs0).wait()
        scale(rows_v0, ew_v0)

        @pl.when(jj > 0)
        def _():
            pltpu.make_async_copy(dst_hbm.at[w, j0], dst_v.at[0], ds0).wait()
        pltpu.async_copy(rows_v0, acc_s.at[dst_v.at[0]], ss0, add=True)

        @pl.when(j0 + 2 < nch)
        def _():
            pltpu.async_copy(src_hbm.at[w, j0 + 2], src_v0, is0)
            pltpu.async_copy(ew_hbm.at[w, j0 + 2], ew_v0, es0)

        @pl.when(j1 + 1 < nch)
        def _():
            pltpu.make_async_copy(src_hbm.at[w, j0 + 2], src_v0, is0).wait()
            pltpu.make_async_copy(ew_hbm.at[w, j0 + 2], ew_v0, es0).wait()
        pltpu.make_async_copy(rows_v0, acc_s.at[dst_v.at[0]], ss0).wait()

        @pl.when(j0 + 2 < nch)
        def _():
            pltpu.async_copy(dst_hbm.at[w, j0 + 2], dst_v.at[0], ds0)
            pltpu.async_copy(h_hbm.at[src_v0], rows_v0, gs0)
        pltpu.make_async_copy(h_hbm.at[src_v1], rows_v1, gs1).wait()
        scale(rows_v1, ew_v1)
        pltpu.make_async_copy(dst_hbm.at[w, j1], dst_v.at[1], ds1).wait()
        pltpu.async_copy(rows_v1, acc_s.at[dst_v.at[1]], ss1, add=True)

        @pl.when(j1 + 2 < nch)
        def _():
            pltpu.async_copy(src_hbm.at[w, j1 + 2], src_v1, is1)
            pltpu.async_copy(ew_hbm.at[w, j1 + 2], ew_v1, es1)
        return carry
    lax.fori_loop(0, nch // 2, step, 0)

    pltpu.make_async_copy(rows_v1, acc_s.at[dst_v.at[1]], ss1).wait()

    plsc.subcore_barrier()
    pltpu.sync_copy(acc_s.at[pl.ds(s * SLAB, SLAB)],
                    out_hbm.at[c, pl.ds(s * SLAB, SLAB)])


def _sc_scatter(h, srcp, dstp, ewp):
    mesh = plsc.VectorSubcoreMesh(core_axis_name="c", subcore_axis_name="s",
                                  num_cores=NC, num_subcores=NS)
    k = pl.kernel(
        _sc_body,
        out_type=jax.ShapeDtypeStruct((NC, NPAD, D), jnp.float32),
        mesh=mesh,
        scratch_types=[
            pltpu.VMEM((CH,), jnp.int32),
            pltpu.VMEM((CH,), jnp.int32),
            pltpu.VMEM((2, CH), jnp.int32),
            pltpu.VMEM((CH, 16), jnp.float32),
            pltpu.VMEM((CH, 16), jnp.float32),
            pltpu.VMEM((CH, D), jnp.float32),
            pltpu.VMEM((CH, D), jnp.float32),
            pltpu.VMEM_SHARED((NPAD, D), jnp.float32),
            pltpu.SemaphoreType.DMA,
            pltpu.SemaphoreType.DMA,
            pltpu.SemaphoreType.DMA,
            pltpu.SemaphoreType.DMA,
            pltpu.SemaphoreType.DMA,
            pltpu.SemaphoreType.DMA,
            pltpu.SemaphoreType.DMA,
            pltpu.SemaphoreType.DMA,
            pltpu.SemaphoreType.DMA,
            pltpu.SemaphoreType.DMA,
        ],
    )
    return k(h, srcp, dstp, ewp)


def _split_edges(arr):
    a0 = arr[:E0].reshape(NS, NCH0, CH)
    a0 = jnp.pad(a0, ((0, 0), (0, NCHM - NCH0), (0, 0)))
    per1 = (E - E0) // NS
    a1 = arr[E0:].reshape(NS, per1)
    a1 = jnp.pad(a1, ((0, 0), (0, NCH1 * CH - per1))).reshape(NS, NCH1, CH)
    a1 = jnp.pad(a1, ((0, 0), (0, NCHM - NCH1), (0, 0)))
    return jnp.concatenate([a0, a1], axis=0)


def kernel(x, edge_idx, edge_attr, W0, b0, W1, b1, W2, b2):
    srcp = _split_edges(edge_idx[0])
    dstp = _split_edges(edge_idx[1])
    ewp = (jnp.broadcast_to(_split_edges(edge_attr)[..., None],
                            (NW, NCHM, CH, 16))
           + jnp.zeros((), jnp.float32))
    b0r = b0.reshape(1, D)
    b1r = b1.reshape(1, D)
    b2r = b2.reshape(1, D)

    h = _mm(x, W0)
    a = _sc_scatter(h, srcp, dstp, ewp)
    h = _cmm(a, b0r, W1)
    a = _sc_scatter(h, srcp, dstp, ewp)
    h = _cmm(a, b1r, W2)
    a = _sc_scatter(h, srcp, dstp, ewp)
    return _final(a, b2r)
